# baseline (speedup 1.0000x reference)
import functools

import jax
import jax.numpy as jnp
from jax import lax
from jax.experimental import pallas as pl
from jax.experimental.pallas import tpu as pltpu
from jax.experimental.pallas import tpu_sc as plsc

_ROWS = 50000
_COLS = 256
_NSEL = 16
_L = 16
_NC = 2
_NS = 16
_NW = _NC * _NS
_CH = 200
_NCHUNK = _ROWS // _CH
_NITER = -(-_NCHUNK // _NW)


def _sc_body(x_hbm, mean_hbm, scale_hbm, out_hbm,
             spm, headb, tailb, mean_v, scale_v, isem0, isem1, osem0, osem1):
    cid = lax.axis_index("c")
    sid = lax.axis_index("s")
    wid = sid * _NC + cid

    pltpu.sync_copy(mean_hbm, mean_v)
    pltpu.sync_copy(scale_hbm, scale_v)
    mv = mean_v[...]
    rsv = 1.0 / scale_v[...]

    lane = lax.broadcasted_iota(jnp.int32, (_L,), 0)
    perm = (_L - lane) & (_L - 1)
    is0 = lane == 0
    _dnums = lax.GatherDimensionNumbers(
        offset_dims=(), collapsed_slice_dims=(0,), start_index_map=(0,))

    def _permute(v):
        return lax.gather(v, perm[:, None], _dnums, slice_sizes=(1,),
                          mode=lax.GatherScatterMode.PROMISE_IN_BOUNDS)

    base = sid * 2 * _CH
    bufs = (spm.at[pl.ds(base, _CH)], spm.at[pl.ds(base + _CH, _CH)])
    isems = (isem0, isem1)
    osems = (osem0, osem1)

    def row0(i):
        return (wid + i * _NW) * _CH

    def compute(s):
        pltpu.sync_copy(bufs[s].at[:, pl.ds(0, _NSEL)], headb)
        pltpu.sync_copy(bufs[s].at[:, pl.ds(_COLS - _L, _L)], tailb)

        def fix_row(r, carry):
            head = headb[r, pl.ds(0, _L)]
            tail = tailb[r, pl.ds(0, _L)]
            g = _permute(tail)
            src = jnp.where(is0, head, g)
            headb[r, pl.ds(0, _L)] = (src - mv) * rsv
            return carry

        lax.fori_loop(0, _CH, fix_row, 0)
        pltpu.sync_copy(headb, bufs[s].at[:, pl.ds(0, _NSEL)])

    def _wait_in(s):
        pltpu.make_async_copy(x_hbm.at[pl.ds(0, _CH)], bufs[s], isems[s]).wait()

    def _wait_out(s):
        pltpu.make_async_copy(bufs[s], out_hbm.at[pl.ds(0, _CH)],
                              osems[s]).wait()

    last = _NITER - 1
    has_last = wid < _NCHUNK - last * _NW

    pltpu.async_copy(x_hbm.at[pl.ds(row0(0), _CH)], bufs[0], isems[0])
    for i in range(last):
        s = i & 1
        if i >= 1:
            _wait_out(1 - s)
        if i + 1 < last:
            pltpu.async_copy(x_hbm.at[pl.ds(row0(i + 1), _CH)],
                             bufs[1 - s], isems[1 - s])
        elif i + 1 == last:
            @pl.when(has_last)
            def _():
                pltpu.async_copy(x_hbm.at[pl.ds(row0(last), _CH)],
                                 bufs[1 - s], isems[1 - s])
        _wait_in(s)
        compute(s)
        pltpu.async_copy(bufs[s], out_hbm.at[pl.ds(row0(i), _CH)], osems[s])
    _wait_out((last - 1) & 1)

    @pl.when(has_last)
    def _():
        s = last & 1
        _wait_in(s)
        compute(s)
        pltpu.async_copy(bufs[s], out_hbm.at[pl.ds(row0(last), _CH)], osems[s])
        _wait_out(s)


@functools.partial(jax.jit, static_argnames=())
def _sc_transform(X, mean, scale):
    mesh = plsc.VectorSubcoreMesh(core_axis_name="c", subcore_axis_name="s")
    return pl.kernel(
        _sc_body,
        out_type=jax.ShapeDtypeStruct((_ROWS, _COLS), jnp.float32),
        mesh=mesh,
        compiler_params=pltpu.CompilerParams(use_tc_tiling_on_sc=False),
        scratch_types=[
            pltpu.MemorySpace.VMEM_SHARED((_NS * 2 * _CH, _COLS), jnp.float32),
            pltpu.VMEM((_CH, _NSEL), jnp.float32),
            pltpu.VMEM((_CH, _L), jnp.float32),
            pltpu.VMEM((_L,), jnp.float32),
            pltpu.VMEM((_L,), jnp.float32),
            pltpu.SemaphoreType.DMA,
            pltpu.SemaphoreType.DMA,
            pltpu.SemaphoreType.DMA,
            pltpu.SemaphoreType.DMA,
        ],
    )(X, mean, scale)


def kernel(X, mean, scale, inds):
    del inds
    return _sc_transform(X, mean, scale)


# speedup vs baseline: 1.7189x; 1.7189x over previous
import functools

import jax
import jax.numpy as jnp
from jax import lax
from jax.experimental import pallas as pl
from jax.experimental.pallas import tpu as pltpu
from jax.experimental.pallas import tpu_sc as plsc

_ROWS = 50000
_COLS = 256
_NSEL = 16
_L = 16
_NC = 2
_NS = 16
_NW = _NC * _NS
_CH = 80
_NCHUNK = _ROWS // _CH
_NITER = -(-_NCHUNK // _NW)


def _sc_body(x_hbm, mean_hbm, scale_hbm, out_hbm,
             spm, headb, tailb, mean_v, scale_v, isem0, isem1, osem0, osem1):
    cid = lax.axis_index("c")
    sid = lax.axis_index("s")
    wid = sid * _NC + cid

    pltpu.sync_copy(mean_hbm, mean_v)
    pltpu.sync_copy(scale_hbm, scale_v)
    mv = mean_v[...]
    rsv = 1.0 / scale_v[...]

    lane = lax.broadcasted_iota(jnp.int32, (_L,), 0)
    perm = (_L - lane) & (_L - 1)
    is0 = lane == 0
    _dnums = lax.GatherDimensionNumbers(
        offset_dims=(), collapsed_slice_dims=(0,), start_index_map=(0,))

    def _permute(v):
        return lax.gather(v, perm[:, None], _dnums, slice_sizes=(1,),
                          mode=lax.GatherScatterMode.PROMISE_IN_BOUNDS)

    base = sid * 2 * _CH
    bufs = (spm.at[pl.ds(base, _CH)], spm.at[pl.ds(base + _CH, _CH)])
    isems = (isem0, isem1)
    osems = (osem0, osem1)

    def row0(i):
        return (wid + i * _NW) * _CH

    def compute(s):
        pltpu.sync_copy(bufs[s].at[:, pl.ds(0, 128)], headb)
        pltpu.sync_copy(bufs[s].at[:, pl.ds(128, 128)], tailb)

        def fix_row(r, carry):
            head = headb[r, pl.ds(0, _L)]
            tail = tailb[r, pl.ds(112, _L)]
            g = _permute(tail)
            src = jnp.where(is0, head, g)
            headb[r, pl.ds(0, _L)] = (src - mv) * rsv
            return carry

        lax.fori_loop(0, _CH, fix_row, 0)
        pltpu.sync_copy(headb, bufs[s].at[:, pl.ds(0, 128)])

    def _wait_in(s):
        pltpu.make_async_copy(x_hbm.at[pl.ds(0, _CH)], bufs[s], isems[s]).wait()

    def _wait_out(s):
        pltpu.make_async_copy(bufs[s], out_hbm.at[pl.ds(0, _CH)],
                              osems[s]).wait()

    last = _NITER - 1
    has_last = wid < _NCHUNK - last * _NW

    pltpu.async_copy(x_hbm.at[pl.ds(row0(0), _CH)], bufs[0], isems[0])
    for i in range(last):
        s = i & 1
        if i >= 1:
            _wait_out(1 - s)
        if i + 1 < last:
            pltpu.async_copy(x_hbm.at[pl.ds(row0(i + 1), _CH)],
                             bufs[1 - s], isems[1 - s])
        elif i + 1 == last:
            @pl.when(has_last)
            def _():
                pltpu.async_copy(x_hbm.at[pl.ds(row0(last), _CH)],
                                 bufs[1 - s], isems[1 - s])
        _wait_in(s)
        compute(s)
        pltpu.async_copy(bufs[s], out_hbm.at[pl.ds(row0(i), _CH)], osems[s])
    _wait_out((last - 1) & 1)

    @pl.when(has_last)
    def _():
        s = last & 1
        _wait_in(s)
        compute(s)
        pltpu.async_copy(bufs[s], out_hbm.at[pl.ds(row0(last), _CH)], osems[s])
        _wait_out(s)


@functools.partial(jax.jit, static_argnames=())
def _sc_transform(X, mean, scale):
    mesh = plsc.VectorSubcoreMesh(core_axis_name="c", subcore_axis_name="s")
    return pl.kernel(
        _sc_body,
        out_type=jax.ShapeDtypeStruct((_ROWS, _COLS), jnp.float32),
        mesh=mesh,
        scratch_types=[
            pltpu.MemorySpace.VMEM_SHARED((_NS * 2 * _CH, _COLS), jnp.float32),
            pltpu.VMEM((_CH, 128), jnp.float32),
            pltpu.VMEM((_CH, 128), jnp.float32),
            pltpu.VMEM((_L,), jnp.float32),
            pltpu.VMEM((_L,), jnp.float32),
            pltpu.SemaphoreType.DMA,
            pltpu.SemaphoreType.DMA,
            pltpu.SemaphoreType.DMA,
            pltpu.SemaphoreType.DMA,
        ],
    )(X, mean, scale)


def kernel(X, mean, scale, inds):
    del inds
    return _sc_transform(X, mean, scale)


# speedup vs baseline: 2.7428x; 1.5957x over previous
import functools

import jax
import jax.numpy as jnp
from jax import lax
from jax.experimental import pallas as pl
from jax.experimental.pallas import tpu as pltpu
from jax.experimental.pallas import tpu_sc as plsc

_ROWS = 50000
_COLS = 256
_NSEL = 16
_L = 16
_NC = 2
_NS = 16
_NW = _NC * _NS
_CH = 200
_NCHUNK = _ROWS // _CH
_NITER = -(-_NCHUNK // _NW)


def _sc_body(x_hbm, mean_hbm, scale_hbm, out_hbm,
             buf0, buf1, mean_v, scale_v, isem0, isem1, osem0, osem1):
    wid = lax.axis_index("s") * _NC + lax.axis_index("c")

    isem_first = isem0
    pltpu.async_copy(x_hbm.at[pl.ds(wid * _CH, _CH)], buf0, isem_first)

    pltpu.sync_copy(mean_hbm, mean_v)
    pltpu.sync_copy(scale_hbm, scale_v)
    mv = mean_v[...]
    rsv = 1.0 / scale_v[...]

    lane = lax.broadcasted_iota(jnp.int32, (_L,), 0)
    perm = (_L - lane) & (_L - 1)
    is0 = lane == 0
    _dnums = lax.GatherDimensionNumbers(
        offset_dims=(), collapsed_slice_dims=(0,), start_index_map=(0,))

    def _permute(v):
        return lax.gather(v, perm[:, None], _dnums, slice_sizes=(1,),
                          mode=lax.GatherScatterMode.PROMISE_IN_BOUNDS)

    bufs = (buf0, buf1)
    isems = (isem0, isem1)
    osems = (osem0, osem1)

    def row0(i):
        return (wid + i * _NW) * _CH

    def compute(buf):
        def fix_row(r, carry):
            head = buf[r, pl.ds(0, _L)]
            tail = buf[r, pl.ds(_COLS - _L, _L)]
            g = _permute(tail)
            src = jnp.where(is0, head, g)
            buf[r, pl.ds(0, _L)] = (src - mv) * rsv
            return carry

        lax.fori_loop(0, _CH, fix_row, 0)

    last = _NITER - 1
    has_last = wid < _NCHUNK - last * _NW

    in_d = [None] * _NITER
    out_d = [None] * _NITER
    in_d[0] = pltpu.make_async_copy(
        x_hbm.at[pl.ds(row0(0), _CH)], bufs[0], isems[0])
    for i in range(last):
        s = i & 1
        if i >= 1:
            out_d[i - 1].wait()
        if i + 1 < last:
            in_d[i + 1] = pltpu.async_copy(
                x_hbm.at[pl.ds(row0(i + 1), _CH)], bufs[1 - s], isems[1 - s])
        elif i + 1 == last:
            @pl.when(has_last)
            def _():
                pltpu.async_copy(
                    x_hbm.at[pl.ds(row0(last), _CH)], bufs[1 - s], isems[1 - s])
        in_d[i].wait()
        compute(bufs[s])
        out_d[i] = pltpu.async_copy(
            bufs[s], out_hbm.at[pl.ds(row0(i), _CH)], osems[s])
    out_d[last - 1].wait()

    @pl.when(has_last)
    def _():
        s = last & 1
        pltpu.make_async_copy(
            x_hbm.at[pl.ds(row0(last), _CH)], bufs[s], isems[s]).wait()
        compute(bufs[s])
        pltpu.async_copy(
            bufs[s], out_hbm.at[pl.ds(row0(last), _CH)], osems[s]).wait()


@functools.partial(jax.jit, static_argnames=())
def _sc_transform(X, mean, scale):
    mesh = plsc.VectorSubcoreMesh(core_axis_name="c", subcore_axis_name="s")
    return pl.kernel(
        _sc_body,
        out_type=jax.ShapeDtypeStruct((_ROWS, _COLS), jnp.float32),
        mesh=mesh,
        scratch_types=[
            pltpu.VMEM((_CH, _COLS), jnp.float32),
            pltpu.VMEM((_CH, _COLS), jnp.float32),
            pltpu.VMEM((_L,), jnp.float32),
            pltpu.VMEM((_L,), jnp.float32),
            pltpu.SemaphoreType.DMA,
            pltpu.SemaphoreType.DMA,
            pltpu.SemaphoreType.DMA,
            pltpu.SemaphoreType.DMA,
        ],
    )(X, mean, scale)


def kernel(X, mean, scale, inds):
    del inds
    return _sc_transform(X, mean, scale)
